# baseline (speedup 1.0000x reference)
import functools

import jax
import jax.numpy as jnp
from jax import lax
from jax.experimental import pallas as pl
from jax.experimental.pallas import tpu as pltpu
from jax.experimental.pallas import tpu_sc as plsc

N_NODES = 10000
D_IN = 128
D_OUT = 128
N_EDGES = 320000

NC = 2
NS = 16
NW = NC * NS
CHUNK = 128
DW = 144
CHUNKS_PER_TILE = 80
E_PAD = NW * CHUNK * CHUNKS_PER_TILE
ROWS_PER_TILE_ACC = 632
N_ACC = NS * ROWS_PER_TILE_ACC
DUMMY_ROW = N_ACC - 1

ROW_BLOCK = 1000


def _prep_body(x_ref, wl_ref, h_ref):
    h = lax.dot_general(
        x_ref[...], wl_ref[...], (((1,), (1,)), ((), ())),
        preferred_element_type=jnp.float32,
        precision=lax.Precision.HIGHEST,
    )
    h_ref[:, pl.ds(0, D_IN)] = h
    col = lax.broadcasted_iota(jnp.int32, (ROW_BLOCK, DW - D_IN), 1)
    h_ref[:, pl.ds(D_IN, DW - D_IN)] = jnp.where(col == 0, 1.0, 0.0).astype(
        jnp.float32
    )


def _final_body(x_ref, wr_ref, a0_ref, a1_ref, bl_ref, br_ref, out_ref):
    s = a0_ref[...] + a1_ref[...]
    feats = s[:, :D_IN]
    cnt = s[:, D_IN:D_IN + 1]
    agg = feats / jnp.maximum(cnt, 1.0)
    dense = lax.dot_general(
        x_ref[...], wr_ref[...], (((1,), (1,)), ((), ())),
        preferred_element_type=jnp.float32,
        precision=lax.Precision.HIGHEST,
    )
    out_ref[...] = agg + dense + bl_ref[...] + br_ref[...]


def _sc_agg(h, sd_idx):
    mesh = plsc.VectorSubcoreMesh(core_axis_name="c", subcore_axis_name="s")

    @functools.partial(
        pl.kernel,
        mesh=mesh,
        compiler_params=pltpu.CompilerParams(use_tc_tiling_on_sc=False),
        out_type=jax.ShapeDtypeStruct((NC, N_ACC, DW), jnp.float32),
        scratch_types=[
            pltpu.VMEM((2, CHUNK), jnp.int32),
            pltpu.VMEM((2, CHUNK), jnp.int32),
            pltpu.VMEM((2, CHUNK), jnp.int32),
            pltpu.VMEM((2, CHUNK), jnp.int32),
            pltpu.VMEM((CHUNK, DW), jnp.float32),
            pltpu.VMEM((CHUNK, DW), jnp.float32),
            pltpu.VMEM_SHARED((N_ACC, DW), jnp.float32),
            pltpu.SemaphoreType.DMA,
            pltpu.SemaphoreType.DMA,
            pltpu.SemaphoreType.DMA,
            pltpu.SemaphoreType.DMA,
            pltpu.SemaphoreType.DMA,
            pltpu.SemaphoreType.DMA,
            pltpu.SemaphoreType.DMA,
            pltpu.SemaphoreType.DMA,
        ],
    )
    def k(h_hbm, sd_hbm, acc_hbm, ib0, ib1, ib2, ib3, gbuf0, gbuf1, acc_sh,
          sem_i0, sem_i1, sem_i2, sem_i3, sem_g0, sem_g1, sem_s0, sem_s1):
        c = lax.axis_index("c")
        s = lax.axis_index("s")
        wid = c * NS + s
        ibs = (ib0, ib1, ib2, ib3)
        sem_is = (sem_i0, sem_i1, sem_i2, sem_i3)

        def fetch_idx(chunk, setno):
            pltpu.async_copy(sd_hbm.at[wid, chunk], ibs[setno], sem_is[setno])

        def wait_idx(setno):
            pltpu.make_async_copy(
                sd_hbm.at[wid, 0], ibs[setno], sem_is[setno]
            ).wait()

        def gather(chunk_set, buf, sem):
            pltpu.async_copy(h_hbm.at[ibs[chunk_set].at[0]], buf, sem)

        def wait_gather(buf, sem):
            pltpu.make_async_copy(h_hbm.at[ib0.at[0]], buf, sem).wait()

        def scatter(chunk_set, buf, sem):
            pltpu.async_copy(buf, acc_sh.at[ibs[chunk_set].at[1]], sem,
                             add=True)

        def wait_scatter(buf, sem):
            pltpu.make_async_copy(buf, acc_sh.at[ib0.at[1]], sem).wait()

        @pl.loop(0, CHUNK)
        def _(r):
            for j in range(DW // 16):
                gbuf0[r, pl.ds(j * 16, 16)] = jnp.zeros((16,), jnp.float32)

        base = s * ROWS_PER_TILE_ACC
        for kk in range(4):
            pltpu.sync_copy(gbuf0, acc_sh.at[pl.ds(base + kk * CHUNK, CHUNK)])
        pltpu.sync_copy(
            gbuf0.at[pl.ds(0, ROWS_PER_TILE_ACC - 4 * CHUNK)],
            acc_sh.at[pl.ds(base + 4 * CHUNK, ROWS_PER_TILE_ACC - 4 * CHUNK)],
        )

        plsc.subcore_barrier()

        for j in range(4):
            fetch_idx(j, j)
        wait_idx(0)
        gather(0, gbuf0, sem_g0)
        wait_idx(1)
        gather(1, gbuf1, sem_g1)

        @pl.loop(0, CHUNKS_PER_TILE - 4, step=4)
        def _(i):
            wait_gather(gbuf0, sem_g0)
            scatter(0, gbuf0, sem_s0)
            wait_gather(gbuf1, sem_g1)
            scatter(1, gbuf1, sem_s1)
            wait_idx(2)
            wait_scatter(gbuf0, sem_s0)
            gather(2, gbuf0, sem_g0)
            fetch_idx(i + 4, 0)
            wait_idx(3)
            wait_scatter(gbuf1, sem_s1)
            gather(3, gbuf1, sem_g1)
            fetch_idx(i + 5, 1)
            wait_gather(gbuf0, sem_g0)
            scatter(2, gbuf0, sem_s0)
            wait_gather(gbuf1, sem_g1)
            scatter(3, gbuf1, sem_s1)
            wait_idx(0)
            wait_scatter(gbuf0, sem_s0)
            gather(0, gbuf0, sem_g0)
            fetch_idx(i + 6, 2)
            wait_idx(1)
            wait_scatter(gbuf1, sem_s1)
            gather(1, gbuf1, sem_g1)
            fetch_idx(i + 7, 3)

        wait_gather(gbuf0, sem_g0)
        scatter(0, gbuf0, sem_s0)
        wait_gather(gbuf1, sem_g1)
        scatter(1, gbuf1, sem_s1)
        wait_idx(2)
        wait_scatter(gbuf0, sem_s0)
        gather(2, gbuf0, sem_g0)
        wait_idx(3)
        wait_scatter(gbuf1, sem_s1)
        gather(3, gbuf1, sem_g1)
        wait_gather(gbuf0, sem_g0)
        scatter(2, gbuf0, sem_s0)
        wait_gather(gbuf1, sem_g1)
        scatter(3, gbuf1, sem_s1)
        wait_scatter(gbuf0, sem_s0)
        wait_scatter(gbuf1, sem_s1)

        plsc.subcore_barrier()

        pltpu.sync_copy(
            acc_sh.at[pl.ds(base, ROWS_PER_TILE_ACC)],
            acc_hbm.at[c, pl.ds(base, ROWS_PER_TILE_ACC)],
        )

    return k(h, sd_idx)


def kernel(x, edge_index, W_l, b_l, W_r, b_r):
    dst = edge_index[0].astype(jnp.int32)
    src = edge_index[1].astype(jnp.int32)
    n_pad = E_PAD - N_EDGES
    src_p = jnp.concatenate([src, jnp.zeros((n_pad,), jnp.int32)])
    dst_p = jnp.concatenate([dst, jnp.full((n_pad,), DUMMY_ROW, jnp.int32)])
    src_t = src_p.reshape(NW, CHUNKS_PER_TILE, CHUNK)
    dst_t = dst_p.reshape(NW, CHUNKS_PER_TILE, CHUNK)
    sd_t = jnp.stack([src_t, dst_t], axis=2)

    grid = N_NODES // ROW_BLOCK
    h = pl.pallas_call(
        _prep_body,
        grid=(grid,),
        in_specs=[
            pl.BlockSpec((ROW_BLOCK, D_IN), lambda i: (i, 0)),
            pl.BlockSpec((D_OUT, D_IN), lambda i: (0, 0)),
        ],
        out_specs=pl.BlockSpec((ROW_BLOCK, DW), lambda i: (i, 0)),
        out_shape=jax.ShapeDtypeStruct((N_NODES, DW), jnp.float32),
    )(x, W_l)

    acc = _sc_agg(h, sd_t)

    out = pl.pallas_call(
        _final_body,
        grid=(grid,),
        in_specs=[
            pl.BlockSpec((ROW_BLOCK, D_IN), lambda i: (i, 0)),
            pl.BlockSpec((D_OUT, D_IN), lambda i: (0, 0)),
            pl.BlockSpec((ROW_BLOCK, DW), lambda i: (i, 0)),
            pl.BlockSpec((ROW_BLOCK, DW), lambda i: (i, 0)),
            pl.BlockSpec((1, D_OUT), lambda i: (0, 0)),
            pl.BlockSpec((1, D_OUT), lambda i: (0, 0)),
        ],
        out_specs=pl.BlockSpec((ROW_BLOCK, D_OUT), lambda i: (i, 0)),
        out_shape=jax.ShapeDtypeStruct((N_NODES, D_OUT), jnp.float32),
    )(x, W_r, acc[0], acc[1], b_l.reshape(1, D_OUT), b_r.reshape(1, D_OUT))
    return out
